# baseline (speedup 1.0000x reference)
import functools

import jax
import jax.numpy as jnp
from jax import lax
from jax.experimental import pallas as pl
from jax.experimental.pallas import tpu as pltpu
from jax.experimental.pallas import tpu_sc as plsc

_K = 512
_ITERS = 10
_D = 256
_BLK = 512
_NBLK = 32
_N = _NBLK * _BLK
_NW = 32
_SEG_W = _K // _NW
_L = 16
_NV = _D // _L


def _assign_body(x_ref, c_ref, csq_ref, xsq_ref, asg_ref, cnt_ref, gr_ref):
    b = pl.program_id(0)

    @pl.when(b == 0)
    def _():
        cnt_ref[...] = jnp.zeros_like(cnt_ref)

    xb = x_ref[...]
    s = jax.lax.dot_general(xb, c_ref[...], (((1,), (1,)), ((), ())),
                            preferred_element_type=jnp.float32)
    d2 = (xsq_ref[...] - 2.0 * s) + csq_ref[...]
    m = jnp.min(d2, axis=1, keepdims=True)
    col = jax.lax.broadcasted_iota(jnp.int32, d2.shape, 1)
    idx = jnp.min(jnp.where(d2 == m, col, _K), axis=1, keepdims=True)
    asg_ref[...] = idx.astype(jnp.float32)
    h = (col == idx).astype(jnp.float32)

    row = jax.lax.broadcasted_iota(jnp.int32, (_BLK, _BLK), 0)
    colb = jax.lax.broadcasted_iota(jnp.int32, (_BLK, _BLK), 1)
    ltri = (colb <= row).astype(jnp.float32)
    rmat = jax.lax.dot_general(ltri, h, (((1,), (0,)), ((), ())),
                               preferred_element_type=jnp.float32)
    rank_in = jnp.sum(rmat * h, axis=1, keepdims=True)
    prev = cnt_ref[...]
    phi = jnp.floor(prev / 128.0)
    plo = prev - 128.0 * phi
    pc = 128.0 * jax.lax.dot_general(h, phi, (((1,), (0,)), ((), ())),
                                     preferred_element_type=jnp.float32) \
        + jax.lax.dot_general(h, plo, (((1,), (0,)), ((), ())),
                              preferred_element_type=jnp.float32)
    gr_ref[...] = pc + rank_in - 1.0

    cnt_ref[...] += jax.lax.dot_general(
        h, jnp.ones((h.shape[0], 1), jnp.float32),
        (((0,), (0,)), ((), ())), preferred_element_type=jnp.float32)


def _assign_call(X, C, csq, x_sq):
    return pl.pallas_call(
        _assign_body,
        grid=(_NBLK,),
        in_specs=[pl.BlockSpec((_BLK, _D), lambda b: (b, 0)),
                  pl.BlockSpec((_K, _D), lambda b: (0, 0)),
                  pl.BlockSpec((1, _K), lambda b: (0, 0)),
                  pl.BlockSpec((_BLK, 1), lambda b: (b, 0))],
        out_specs=[pl.BlockSpec((_BLK, 1), lambda b: (b, 0)),
                   pl.BlockSpec((_K, 1), lambda b: (0, 0)),
                   pl.BlockSpec((_BLK, 1), lambda b: (b, 0))],
        out_shape=[jax.ShapeDtypeStruct((_N, 1), jnp.float32),
                   jax.ShapeDtypeStruct((_K, 1), jnp.float32),
                   jax.ShapeDtypeStruct((_N, 1), jnp.float32)],
        compiler_params=pltpu.CompilerParams(
            dimension_semantics=("arbitrary",)),
    )(X, C, csq, x_sq)


def _dest_body(asg_ref, gr_ref, shi_ref, slo_ref, dest_ref):
    col = jax.lax.broadcasted_iota(jnp.int32, (_BLK, _K), 1)
    h = (col == asg_ref[...].astype(jnp.int32)).astype(jnp.float32)
    st = 128.0 * jax.lax.dot_general(h, shi_ref[...], (((1,), (0,)), ((), ())),
                                     preferred_element_type=jnp.float32) \
        + jax.lax.dot_general(h, slo_ref[...], (((1,), (0,)), ((), ())),
                              preferred_element_type=jnp.float32)
    dest_ref[...] = st + gr_ref[...]


def _dest_call(asg, gr, shi, slo):
    return pl.pallas_call(
        _dest_body,
        grid=(_NBLK,),
        in_specs=[pl.BlockSpec((_BLK, 1), lambda b: (b, 0)),
                  pl.BlockSpec((_BLK, 1), lambda b: (b, 0)),
                  pl.BlockSpec((_K, 1), lambda b: (0, 0)),
                  pl.BlockSpec((_K, 1), lambda b: (0, 0))],
        out_specs=pl.BlockSpec((_BLK, 1), lambda b: (b, 0)),
        out_shape=jax.ShapeDtypeStruct((_N, 1), jnp.float32),
    )(asg, gr, shi, slo)


_MESH = plsc.VectorSubcoreMesh(core_axis_name="c", subcore_axis_name="s")
_SEG_T = _K // 32
_ROW_T = _N // 16


def _chunk_index(q):
    half = q // 8192
    r = q - half * 8192
    ci = jnp.where(r < 5600, r // 560,
                   jnp.where(r < 7840, 10 + (r - 5600) // 448, 15))
    return half * 16 + ci


def _next_boundary(q):
    half = (q // 8192) * 8192
    r = q - half
    nb560 = half + (r // 560 + 1) * 560
    nb448 = half + 5600 + ((r - 5600) // 448 + 1) * 448
    nb = jnp.where(r < 5600, nb560, jnp.where(r < 7840, nb448, half + 8192))
    return nb


def _order_sc(dest_hbm, order_hbm, dest_rows, vals_v, sem):
    wid = lax.axis_index("s") * 2 + lax.axis_index("c")
    base = wid * (_N // _NW)
    for k in range(4):
        pltpu.sync_copy(dest_hbm.at[pl.ds(base + 128 * k, 128)],
                        dest_rows.at[k])
    for k in range(4):
        for mm in range(8):
            vals_v[pl.ds(16 * mm, 16)] = (
                lax.iota(jnp.int32, 16) + (base + 128 * k + 16 * mm))
        pltpu.async_copy(vals_v, order_hbm.at[dest_rows.at[k]], sem).wait()


def _order_call(dest_i):
    kfn = functools.partial(
        pl.kernel, mesh=_MESH,
        out_type=jax.ShapeDtypeStruct((_N,), jnp.int32),
        scratch_types=[pltpu.VMEM((4, 128), jnp.int32),
                       pltpu.VMEM((128,), jnp.int32),
                       pltpu.SemaphoreType.DMA],
    )(_order_sc)
    return kfn(dest_i)


def _segsum_sc(x_hbm, order_hbm, st_hbm, en_hbm, sums_hbm,
               oidx_v, rows0_v, rows1_v, stage_v, sv, ev, sem0, sem1):
    wid = lax.axis_index("s") * 2 + lax.axis_index("c")
    pltpu.sync_copy(order_hbm, oidx_v)
    pltpu.sync_copy(st_hbm.at[pl.ds(wid * _SEG_T * 8, _SEG_T * 8)], sv)
    pltpu.sync_copy(en_hbm.at[pl.ds(wid * _SEG_T * 8, _SEG_T * 8)], ev)
    zero = jnp.zeros((_L,), jnp.float32)

    def dma_off(q, t):
        off = q + 120 * t
        return jnp.minimum((off // 8) * 8, _N - 128)

    def start(buf_sem, q, t):
        rows, sem = buf_sem
        return pltpu.async_copy(
            x_hbm.at[oidx_v.at[pl.ds(dma_off(q, t), 128)]], rows, sem)

    def seg_body(j, _carry):
        seg_s = sv[pl.ds(8 * j, 1)][0]
        seg_e = ev[pl.ds(8 * j, 1)][0]

        def accum(rows, q, t, pe, acc):
            off = q + 120 * t
            m0 = off - dma_off(q, t)
            n = jnp.clip(pe - off, 0, 120)

            def member_body(mm, a):
                return tuple(
                    a[v] + rows[mm, pl.ds(_L * v, _L)]
                    for v in range(_NV))

            return lax.fori_loop(m0, m0 + n, member_body, acc)

        def piece_body(pp, carry):
            q = carry[0]
            tot = carry[1:]
            pe = jnp.minimum(_next_boundary(q), seg_e)
            nb = (pe - q + 119) // 120

            start((rows0_v, sem0), q, 0)

            def body2(b2, bc):
                acc = bc
                t0 = 2 * b2
                start((rows1_v, sem1), q, t0 + 1)
                pltpu.make_async_copy(
                    x_hbm.at[oidx_v.at[pl.ds(dma_off(q, t0), 128)]],
                    rows0_v, sem0).wait()
                acc = accum(rows0_v, q, t0, pe, acc)
                start((rows0_v, sem0), q, t0 + 2)
                pltpu.make_async_copy(
                    x_hbm.at[oidx_v.at[pl.ds(dma_off(q, t0 + 1), 128)]],
                    rows1_v, sem1).wait()
                acc = accum(rows1_v, q, t0 + 1, pe, acc)
                return acc

            bfin = lax.fori_loop(0, (nb + 1) // 2, body2, (zero,) * _NV)
            pltpu.make_async_copy(
                x_hbm.at[oidx_v.at[pl.ds(dma_off(q, 0), 128)]],
                rows0_v, sem0).wait()
            tot = tuple(tot[v] + bfin[v] for v in range(_NV))
            return (pe,) + tot

        npieces = jnp.where(
            seg_e > seg_s,
            _chunk_index(seg_e - 1) - _chunk_index(seg_s) + 1, 0)
        fin = lax.fori_loop(0, npieces, piece_body,
                            (seg_s,) + (zero,) * _NV)
        for v in range(_NV):
            stage_v[j, pl.ds(_L * v, _L)] = fin[1 + v]
        return _carry

    lax.fori_loop(0, _SEG_T, seg_body, 0)

    pltpu.sync_copy(stage_v, sums_hbm.at[pl.ds(wid * _SEG_T, _SEG_T)])


def _segsum_call(X, order, starts, ends):
    kfn = functools.partial(
        pl.kernel, mesh=_MESH,
        out_type=jax.ShapeDtypeStruct((_K, _D), jnp.float32),
        scratch_types=[pltpu.VMEM((_N,), jnp.int32),
                       pltpu.VMEM((128, _D), jnp.float32),
                       pltpu.VMEM((128, _D), jnp.float32),
                       pltpu.VMEM((_SEG_T, _D), jnp.float32),
                       pltpu.VMEM((_SEG_T * 8,), jnp.int32),
                       pltpu.VMEM((_SEG_T * 8,), jnp.int32),
                       pltpu.SemaphoreType.DMA,
                       pltpu.SemaphoreType.DMA],
    )(_segsum_sc)
    st8 = jnp.zeros((_K * 8,), jnp.int32).at[::8].set(starts)
    en8 = jnp.zeros((_K * 8,), jnp.int32).at[::8].set(ends)
    return kfn(X, order, st8, en8)


def _onehot_body(asg_ref, g_ref):
    col = jax.lax.broadcasted_iota(jnp.int32, (_BLK, _K), 1)
    idx = asg_ref[...].astype(jnp.int32)
    g_ref[...] = (col == idx).astype(jnp.float32)


def _onehot_call(asg):
    return pl.pallas_call(
        _onehot_body,
        grid=(_NBLK,),
        in_specs=[pl.BlockSpec((_BLK, 1), lambda b: (b, 0))],
        out_specs=pl.BlockSpec((_BLK, _K), lambda b: (b, 0)),
        out_shape=jax.ShapeDtypeStruct((_N, _K), jnp.float32),
    )(asg)


def kernel(X):
    x_sq = (X * X).sum(axis=1, keepdims=True)
    C = X[:_K]
    asg = None
    for t in range(_ITERS):
        csq = (C * C).sum(axis=1)[None, :]
        asg, counts, gr = _assign_call(X, C, csq, x_sq)
        if t == _ITERS - 1:
            break
        cnt_i = counts.astype(jnp.int32).ravel()
        starts = jnp.cumsum(cnt_i) - cnt_i
        ends = starts + cnt_i
        shi = (starts // 128).astype(jnp.float32)[:, None]
        slo = (starts % 128).astype(jnp.float32)[:, None]
        dest = _dest_call(asg, gr, shi, slo)
        order = _order_call(dest.astype(jnp.int32).ravel())
        sums = _segsum_call(X, order, starts, ends)
        C = jnp.where(counts > 0.0, sums / jnp.maximum(counts, 1.0), C)
    return _onehot_call(asg)


# speedup vs baseline: 1.2510x; 1.2510x over previous
import functools

import jax
import jax.numpy as jnp
from jax import lax
from jax.experimental import pallas as pl
from jax.experimental.pallas import tpu as pltpu
from jax.experimental.pallas import tpu_sc as plsc

_K = 512
_ITERS = 10
_D = 256
_BLK = 512
_NBLK = 32
_N = _NBLK * _BLK
_NW = 32
_SEG_W = _K // _NW
_L = 16
_NV = _D // _L


def _assign_body(x_ref, c_ref, csq_ref, xsq_ref, asg_ref, cnt_ref, gr_ref):
    b = pl.program_id(0)

    @pl.when(b == 0)
    def _():
        cnt_ref[...] = jnp.zeros_like(cnt_ref)

    xb = x_ref[...]
    s = jax.lax.dot_general(xb, c_ref[...], (((1,), (1,)), ((), ())),
                            preferred_element_type=jnp.float32)
    d2 = (xsq_ref[...] - 2.0 * s) + csq_ref[...]
    m = jnp.min(d2, axis=1, keepdims=True)
    col = jax.lax.broadcasted_iota(jnp.int32, d2.shape, 1)
    idx = jnp.min(jnp.where(d2 == m, col, _K), axis=1, keepdims=True)
    asg_ref[...] = idx.astype(jnp.float32)
    h = (col == idx).astype(jnp.float32)

    row = jax.lax.broadcasted_iota(jnp.int32, (_BLK, _BLK), 0)
    colb = jax.lax.broadcasted_iota(jnp.int32, (_BLK, _BLK), 1)
    ltri = (colb <= row).astype(jnp.float32)
    rmat = jax.lax.dot_general(ltri, h, (((1,), (0,)), ((), ())),
                               preferred_element_type=jnp.float32)
    rank_in = jnp.sum(rmat * h, axis=1, keepdims=True)
    prev = cnt_ref[...]
    phi = jnp.floor(prev / 128.0)
    plo = prev - 128.0 * phi
    pc = 128.0 * jax.lax.dot_general(h, phi, (((1,), (0,)), ((), ())),
                                     preferred_element_type=jnp.float32) \
        + jax.lax.dot_general(h, plo, (((1,), (0,)), ((), ())),
                              preferred_element_type=jnp.float32)
    gr_ref[...] = pc + rank_in - 1.0

    cnt_ref[...] += jax.lax.dot_general(
        h, jnp.ones((h.shape[0], 1), jnp.float32),
        (((0,), (0,)), ((), ())), preferred_element_type=jnp.float32)


def _assign_call(X, C, csq, x_sq):
    return pl.pallas_call(
        _assign_body,
        grid=(_NBLK,),
        in_specs=[pl.BlockSpec((_BLK, _D), lambda b: (b, 0)),
                  pl.BlockSpec((_K, _D), lambda b: (0, 0)),
                  pl.BlockSpec((1, _K), lambda b: (0, 0)),
                  pl.BlockSpec((_BLK, 1), lambda b: (b, 0))],
        out_specs=[pl.BlockSpec((_BLK, 1), lambda b: (b, 0)),
                   pl.BlockSpec((_K, 1), lambda b: (0, 0)),
                   pl.BlockSpec((_BLK, 1), lambda b: (b, 0))],
        out_shape=[jax.ShapeDtypeStruct((_N, 1), jnp.float32),
                   jax.ShapeDtypeStruct((_K, 1), jnp.float32),
                   jax.ShapeDtypeStruct((_N, 1), jnp.float32)],
        compiler_params=pltpu.CompilerParams(
            dimension_semantics=("arbitrary",)),
    )(X, C, csq, x_sq)


def _dest_body(asg_ref, gr_ref, shi_ref, slo_ref, dest_ref):
    col = jax.lax.broadcasted_iota(jnp.int32, (_BLK, _K), 1)
    h = (col == asg_ref[...].astype(jnp.int32)).astype(jnp.float32)
    st = 128.0 * jax.lax.dot_general(h, shi_ref[...], (((1,), (0,)), ((), ())),
                                     preferred_element_type=jnp.float32) \
        + jax.lax.dot_general(h, slo_ref[...], (((1,), (0,)), ((), ())),
                              preferred_element_type=jnp.float32)
    dest_ref[...] = st + gr_ref[...]


def _dest_call(asg, gr, shi, slo):
    return pl.pallas_call(
        _dest_body,
        grid=(_NBLK,),
        in_specs=[pl.BlockSpec((_BLK, 1), lambda b: (b, 0)),
                  pl.BlockSpec((_BLK, 1), lambda b: (b, 0)),
                  pl.BlockSpec((_K, 1), lambda b: (0, 0)),
                  pl.BlockSpec((_K, 1), lambda b: (0, 0))],
        out_specs=pl.BlockSpec((_BLK, 1), lambda b: (b, 0)),
        out_shape=jax.ShapeDtypeStruct((_N, 1), jnp.float32),
    )(asg, gr, shi, slo)


_MESH = plsc.VectorSubcoreMesh(core_axis_name="c", subcore_axis_name="s")
_SEG_T = _K // 32
_ROW_T = _N // 16


def _chunk_index(q):
    half = q // 8192
    r = q - half * 8192
    ci = jnp.where(r < 5600, r // 560,
                   jnp.where(r < 7840, 10 + (r - 5600) // 448, 15))
    return half * 16 + ci


def _next_boundary(q):
    half = (q // 8192) * 8192
    r = q - half
    nb560 = half + (r // 560 + 1) * 560
    nb448 = half + 5600 + ((r - 5600) // 448 + 1) * 448
    nb = jnp.where(r < 5600, nb560, jnp.where(r < 7840, nb448, half + 8192))
    return nb


def _order_sc(dest_hbm, order_hbm, dest_rows, vals_v, sem):
    wid = lax.axis_index("s") * 2 + lax.axis_index("c")
    base = wid * (_N // _NW)
    for k in range(4):
        pltpu.sync_copy(dest_hbm.at[pl.ds(base + 128 * k, 128)],
                        dest_rows.at[k])
    for k in range(4):
        for mm in range(8):
            vals_v[pl.ds(16 * mm, 16)] = (
                lax.iota(jnp.int32, 16) + (base + 128 * k + 16 * mm))
        pltpu.async_copy(vals_v, order_hbm.at[dest_rows.at[k]], sem).wait()


def _order_call(dest_i):
    kfn = functools.partial(
        pl.kernel, mesh=_MESH,
        out_type=jax.ShapeDtypeStruct((_N,), jnp.int32),
        scratch_types=[pltpu.VMEM((4, 128), jnp.int32),
                       pltpu.VMEM((128,), jnp.int32),
                       pltpu.SemaphoreType.DMA],
    )(_order_sc)
    return kfn(dest_i)


def _segsum_sc(x_hbm, order_hbm, st_hbm, en_hbm, sums_hbm,
               oidx_v, rows_v, stage_v, sv, ev, sem):
    wid = lax.axis_index("s") * 2 + lax.axis_index("c")
    pltpu.sync_copy(order_hbm, oidx_v)
    pltpu.sync_copy(st_hbm.at[pl.ds(wid * _SEG_T, _SEG_T)], sv)
    pltpu.sync_copy(en_hbm.at[pl.ds(wid * _SEG_T, _SEG_T)], ev)
    zero = jnp.zeros((_L,), jnp.float32)

    for j in range(_SEG_T):
        seg_s = sv[pl.ds(j, 1)][0]
        seg_e = ev[pl.ds(j, 1)][0]

        def piece_body(pp, carry):
            q = carry[0]
            tot = carry[1:]
            pe = jnp.minimum(_next_boundary(q), seg_e)

            def batch_body(bb, bc):
                off = bc[0]
                acc = bc[1:]
                off2 = jnp.minimum((off // 8) * 8, _N - 128)
                m0 = off - off2
                n = jnp.minimum(jnp.int32(128) - m0, pe - off)
                pltpu.async_copy(
                    x_hbm.at[oidx_v.at[pl.ds(off2, 128)]], rows_v, sem
                ).wait()

                def member_body(mm, a):
                    return tuple(
                        a[v] + rows_v[mm, pl.ds(_L * v, _L)]
                        for v in range(_NV))

                acc = lax.fori_loop(m0, m0 + n, member_body, acc)
                return (off + n,) + acc

            nbatch = (pe - q + 120) // 121
            bfin = lax.fori_loop(0, nbatch, batch_body,
                                 (q,) + (zero,) * _NV)
            tot = tuple(tot[v] + bfin[1 + v] for v in range(_NV))
            return (pe,) + tot

        npieces = jnp.where(
            seg_e > seg_s,
            _chunk_index(seg_e - 1) - _chunk_index(seg_s) + 1, 0)
        fin = lax.fori_loop(0, npieces, piece_body,
                            (seg_s,) + (zero,) * _NV)
        for v in range(_NV):
            stage_v[j, pl.ds(_L * v, _L)] = fin[1 + v]

    pltpu.sync_copy(stage_v, sums_hbm.at[pl.ds(wid * _SEG_T, _SEG_T)])


def _segsum_call(X, order, starts, ends):
    kfn = functools.partial(
        pl.kernel, mesh=_MESH,
        out_type=jax.ShapeDtypeStruct((_K, _D), jnp.float32),
        scratch_types=[pltpu.VMEM((_N,), jnp.int32),
                       pltpu.VMEM((128, _D), jnp.float32),
                       pltpu.VMEM((_SEG_T, _D), jnp.float32),
                       pltpu.VMEM((_SEG_T,), jnp.int32),
                       pltpu.VMEM((_SEG_T,), jnp.int32),
                       pltpu.SemaphoreType.DMA],
    )(_segsum_sc)
    return kfn(X, order, starts, ends)


def _onehot_body(asg_ref, g_ref):
    col = jax.lax.broadcasted_iota(jnp.int32, (_BLK, _K), 1)
    idx = asg_ref[...].astype(jnp.int32)
    g_ref[...] = (col == idx).astype(jnp.float32)


def _onehot_call(asg):
    return pl.pallas_call(
        _onehot_body,
        grid=(_NBLK,),
        in_specs=[pl.BlockSpec((_BLK, 1), lambda b: (b, 0))],
        out_specs=pl.BlockSpec((_BLK, _K), lambda b: (b, 0)),
        out_shape=jax.ShapeDtypeStruct((_N, _K), jnp.float32),
    )(asg)


def kernel(X):
    x_sq = (X * X).sum(axis=1, keepdims=True)
    C = X[:_K]
    asg = None
    for t in range(_ITERS):
        csq = (C * C).sum(axis=1)[None, :]
        asg, counts, gr = _assign_call(X, C, csq, x_sq)
        if t == _ITERS - 1:
            break
        cnt_i = counts.astype(jnp.int32).ravel()
        starts = jnp.cumsum(cnt_i) - cnt_i
        ends = starts + cnt_i
        shi = (starts // 128).astype(jnp.float32)[:, None]
        slo = (starts % 128).astype(jnp.float32)[:, None]
        dest = _dest_call(asg, gr, shi, slo)
        order = _order_call(dest.astype(jnp.int32).ravel())
        sums = _segsum_call(X, order, starts, ends)
        C = jnp.where(counts > 0.0, sums / jnp.maximum(counts, 1.0), C)
    return _onehot_call(asg)


# speedup vs baseline: 1.2528x; 1.0015x over previous
import functools

import jax
import jax.numpy as jnp
from jax import lax
from jax.experimental import pallas as pl
from jax.experimental.pallas import tpu as pltpu
from jax.experimental.pallas import tpu_sc as plsc

_K = 512
_ITERS = 10
_D = 256
_BLK = 512
_NBLK = 32
_N = _NBLK * _BLK
_NW = 32
_SEG_W = _K // _NW
_L = 16
_NV = _D // _L


def _assign_body(x_ref, c_ref, csq_ref, xsq_ref, asg_ref, cnt_ref, gr_ref):
    b = pl.program_id(0)

    @pl.when(b == 0)
    def _():
        cnt_ref[...] = jnp.zeros_like(cnt_ref)

    xb = x_ref[...]
    s = jax.lax.dot_general(xb, c_ref[...], (((1,), (1,)), ((), ())),
                            preferred_element_type=jnp.float32)
    d2 = (xsq_ref[...] - 2.0 * s) + csq_ref[...]
    m = jnp.min(d2, axis=1, keepdims=True)
    col = jax.lax.broadcasted_iota(jnp.int32, d2.shape, 1)
    idx = jnp.min(jnp.where(d2 == m, col, _K), axis=1, keepdims=True)
    asg_ref[...] = idx.astype(jnp.float32)
    h = (col == idx).astype(jnp.float32)

    row = jax.lax.broadcasted_iota(jnp.int32, (_BLK, _BLK), 0)
    colb = jax.lax.broadcasted_iota(jnp.int32, (_BLK, _BLK), 1)
    ltri = (colb <= row).astype(jnp.float32)
    rmat = jax.lax.dot_general(ltri, h, (((1,), (0,)), ((), ())),
                               preferred_element_type=jnp.float32)
    rank_in = jnp.sum(rmat * h, axis=1, keepdims=True)
    prev = cnt_ref[...]
    phi = jnp.floor(prev / 128.0)
    plo = prev - 128.0 * phi
    pc = 128.0 * jax.lax.dot_general(h, phi, (((1,), (0,)), ((), ())),
                                     preferred_element_type=jnp.float32) \
        + jax.lax.dot_general(h, plo, (((1,), (0,)), ((), ())),
                              preferred_element_type=jnp.float32)
    gr_ref[...] = pc + rank_in - 1.0

    cnt_ref[...] += jax.lax.dot_general(
        h, jnp.ones((h.shape[0], 1), jnp.float32),
        (((0,), (0,)), ((), ())), preferred_element_type=jnp.float32)


def _assign_call(X, C, csq, x_sq):
    return pl.pallas_call(
        _assign_body,
        grid=(_NBLK,),
        in_specs=[pl.BlockSpec((_BLK, _D), lambda b: (b, 0)),
                  pl.BlockSpec((_K, _D), lambda b: (0, 0)),
                  pl.BlockSpec((1, _K), lambda b: (0, 0)),
                  pl.BlockSpec((_BLK, 1), lambda b: (b, 0))],
        out_specs=[pl.BlockSpec((_BLK, 1), lambda b: (b, 0)),
                   pl.BlockSpec((_K, 1), lambda b: (0, 0)),
                   pl.BlockSpec((_BLK, 1), lambda b: (b, 0))],
        out_shape=[jax.ShapeDtypeStruct((_N, 1), jnp.float32),
                   jax.ShapeDtypeStruct((_K, 1), jnp.float32),
                   jax.ShapeDtypeStruct((_N, 1), jnp.float32)],
        compiler_params=pltpu.CompilerParams(
            dimension_semantics=("arbitrary",)),
    )(X, C, csq, x_sq)


def _dest_body(asg_ref, gr_ref, shi_ref, slo_ref, dest_ref):
    col = jax.lax.broadcasted_iota(jnp.int32, (_BLK, _K), 1)
    h = (col == asg_ref[...].astype(jnp.int32)).astype(jnp.float32)
    st = 128.0 * jax.lax.dot_general(h, shi_ref[...], (((1,), (0,)), ((), ())),
                                     preferred_element_type=jnp.float32) \
        + jax.lax.dot_general(h, slo_ref[...], (((1,), (0,)), ((), ())),
                              preferred_element_type=jnp.float32)
    dest_ref[...] = st + gr_ref[...]


def _dest_call(asg, gr, shi, slo):
    return pl.pallas_call(
        _dest_body,
        grid=(_NBLK,),
        in_specs=[pl.BlockSpec((_BLK, 1), lambda b: (b, 0)),
                  pl.BlockSpec((_BLK, 1), lambda b: (b, 0)),
                  pl.BlockSpec((_K, 1), lambda b: (0, 0)),
                  pl.BlockSpec((_K, 1), lambda b: (0, 0))],
        out_specs=pl.BlockSpec((_BLK, 1), lambda b: (b, 0)),
        out_shape=jax.ShapeDtypeStruct((_N, 1), jnp.float32),
    )(asg, gr, shi, slo)


_SEG_T = _K // 32
_ROW_T = _N // 16


def _mesh():
    return plsc.VectorSubcoreMesh(core_axis_name="c", subcore_axis_name="s")


def _chunk_index(q):
    half = q // 8192
    r = q - half * 8192
    ci = jnp.where(r < 5600, r // 560,
                   jnp.where(r < 7840, 10 + (r - 5600) // 448, 15))
    return half * 16 + ci


def _next_boundary(q):
    half = (q // 8192) * 8192
    r = q - half
    nb560 = half + (r // 560 + 1) * 560
    nb448 = half + 5600 + ((r - 5600) // 448 + 1) * 448
    nb = jnp.where(r < 5600, nb560, jnp.where(r < 7840, nb448, half + 8192))
    return nb


def _order_sc(dest_hbm, order_hbm, dest_rows, vals_v, sem):
    wid = lax.axis_index("s") * 2 + lax.axis_index("c")
    base = wid * (_N // _NW)
    for k in range(4):
        pltpu.sync_copy(dest_hbm.at[pl.ds(base + 128 * k, 128)],
                        dest_rows.at[k])
    for k in range(4):
        for mm in range(8):
            vals_v[pl.ds(16 * mm, 16)] = (
                lax.iota(jnp.int32, 16) + (base + 128 * k + 16 * mm))
        pltpu.async_copy(vals_v, order_hbm.at[dest_rows.at[k]], sem).wait()


def _order_call(dest_i):
    kfn = functools.partial(
        pl.kernel, mesh=_mesh(),
        out_type=jax.ShapeDtypeStruct((_N,), jnp.int32),
        scratch_types=[pltpu.VMEM((4, 128), jnp.int32),
                       pltpu.VMEM((128,), jnp.int32),
                       pltpu.SemaphoreType.DMA],
    )(_order_sc)
    return kfn(dest_i)


def _segsum_sc(x_hbm, order_hbm, st_hbm, en_hbm, sums_hbm,
               oidx_v, rows_v, stage_v, sv, ev, sem):
    wid = lax.axis_index("s") * 2 + lax.axis_index("c")
    pltpu.sync_copy(order_hbm, oidx_v)
    pltpu.sync_copy(st_hbm.at[pl.ds(wid * _SEG_T, _SEG_T)], sv)
    pltpu.sync_copy(en_hbm.at[pl.ds(wid * _SEG_T, _SEG_T)], ev)
    zero = jnp.zeros((_L,), jnp.float32)

    for j in range(_SEG_T):
        seg_s = sv[pl.ds(j, 1)][0]
        seg_e = ev[pl.ds(j, 1)][0]

        def piece_body(pp, carry):
            q = carry[0]
            tot = carry[1:]
            pe = jnp.minimum(_next_boundary(q), seg_e)

            def batch_body(bb, bc):
                off = bc[0]
                acc = bc[1:]
                off2 = jnp.minimum((off // 8) * 8, _N - 128)
                m0 = off - off2
                n = jnp.minimum(jnp.int32(128) - m0, pe - off)
                pltpu.async_copy(
                    x_hbm.at[oidx_v.at[pl.ds(off2, 128)]], rows_v, sem
                ).wait()

                def member_body(mm, a):
                    return tuple(
                        a[v] + rows_v[mm, pl.ds(_L * v, _L)]
                        for v in range(_NV))

                acc = lax.fori_loop(m0, m0 + n, member_body, acc)
                return (off + n,) + acc

            nbatch = (pe - q + 120) // 121
            bfin = lax.fori_loop(0, nbatch, batch_body,
                                 (q,) + (zero,) * _NV)
            tot = tuple(tot[v] + bfin[1 + v] for v in range(_NV))
            return (pe,) + tot

        npieces = jnp.where(
            seg_e > seg_s,
            _chunk_index(seg_e - 1) - _chunk_index(seg_s) + 1, 0)
        fin = lax.fori_loop(0, npieces, piece_body,
                            (seg_s,) + (zero,) * _NV)
        for v in range(_NV):
            stage_v[j, pl.ds(_L * v, _L)] = fin[1 + v]

    pltpu.sync_copy(stage_v, sums_hbm.at[pl.ds(wid * _SEG_T, _SEG_T)])


def _segsum_call(X, order, starts, ends):
    kfn = functools.partial(
        pl.kernel, mesh=_mesh(),
        out_type=jax.ShapeDtypeStruct((_K, _D), jnp.float32),
        scratch_types=[pltpu.VMEM((_N,), jnp.int32),
                       pltpu.VMEM((128, _D), jnp.float32),
                       pltpu.VMEM((_SEG_T, _D), jnp.float32),
                       pltpu.VMEM((_SEG_T,), jnp.int32),
                       pltpu.VMEM((_SEG_T,), jnp.int32),
                       pltpu.SemaphoreType.DMA],
    )(_segsum_sc)
    return kfn(X, order, starts, ends)


def _onehot_body(asg_ref, g_ref):
    col = jax.lax.broadcasted_iota(jnp.int32, (_BLK, _K), 1)
    idx = asg_ref[...].astype(jnp.int32)
    g_ref[...] = (col == idx).astype(jnp.float32)


def _onehot_call(asg):
    return pl.pallas_call(
        _onehot_body,
        grid=(_NBLK,),
        in_specs=[pl.BlockSpec((_BLK, 1), lambda b: (b, 0))],
        out_specs=pl.BlockSpec((_BLK, _K), lambda b: (b, 0)),
        out_shape=jax.ShapeDtypeStruct((_N, _K), jnp.float32),
    )(asg)


def kernel(X):
    x_sq = (X * X).sum(axis=1, keepdims=True)
    C = X[:_K]
    asg = None
    for t in range(_ITERS):
        csq = (C * C).sum(axis=1)[None, :]
        asg, counts, gr = _assign_call(X, C, csq, x_sq)
        if t == _ITERS - 1:
            break
        cnt_i = counts.astype(jnp.int32).ravel()
        starts = jnp.cumsum(cnt_i) - cnt_i
        ends = starts + cnt_i
        shi = (starts // 128).astype(jnp.float32)[:, None]
        slo = (starts % 128).astype(jnp.float32)[:, None]
        dest = _dest_call(asg, gr, shi, slo)
        order = _order_call(dest.astype(jnp.int32).ravel())
        sums = _segsum_call(X, order, starts, ends)
        C = jnp.where(counts > 0.0, sums / jnp.maximum(counts, 1.0), C)
    return _onehot_call(asg)
